# Initial kernel scaffold; baseline (speedup 1.0000x reference)
import jax
import jax.numpy as jnp
from jax.experimental import pallas as pl


def kernel(x_nchw, w1_fold, b1_fold, p2):
    raise NotImplementedError("write your pallas kernel here")


# speedup vs baseline: 1.1202x; 1.1202x over previous
import functools

import jax
import jax.numpy as jnp
from jax.experimental import pallas as pl
from jax.experimental.pallas import tpu as pltpu


def _gate_body(p2_ref, x_ref, wa_ref, b1_ref, o_ref, *, nb, c2):
    w = wa_ref[...]
    bias = b1_ref[...]
    t_max = p2_ref[0]
    t_sum = p2_ref[1]
    t_bias = p2_ref[2]
    for b in range(nb):
        x = x_ref[b]
        ya = jnp.dot(w, x, preferred_element_type=jnp.float32)
        x_sum = ya[c2:c2 + 1]
        x_max = jnp.max(x, axis=0, keepdims=True)
        z = x_max * t_max + x_sum * t_sum + t_bias
        z = z * jax.nn.sigmoid(z)
        gate = jax.nn.sigmoid(z)
        y = ya[:c2] + bias
        y = y * jax.nn.sigmoid(y)
        o_ref[b] = (y * gate).astype(o_ref.dtype)


def kernel(x_nchw, w1_fold, b1_fold, p2):
    N, C1, H, W = x_nchw.shape
    C2 = w1_fold.shape[0]
    HW = H * W

    HWp = ((HW + 127) // 128) * 128
    x3 = x_nchw.reshape(N, C1, HW)
    if HWp != HW:
        x3 = jnp.pad(x3, ((0, 0), (0, 0), (0, HWp - HW)))

    wa = jnp.concatenate(
        [w1_fold.astype(jnp.float32),
         jnp.ones((1, C1), jnp.float32),
         jnp.zeros((7, C1), jnp.float32)], axis=0)
    b1_2d = b1_fold.reshape(C2, 1).astype(jnp.float32)
    p2k = jnp.stack([p2[0], p2[1] / jnp.float32(C1), p2[2],
                     jnp.float32(0.0)]).astype(jnp.float32)

    nb = 1
    for cand in (4, 2):
        if N % cand == 0 and cand * (C1 + C2) * HWp * 4 <= (12 << 20):
            nb = cand
            break

    grid = (N // nb,)
    cparams = pltpu.CompilerParams(
        dimension_semantics=("parallel",),
        vmem_limit_bytes=int(56 << 20))

    out3 = pl.pallas_call(
        functools.partial(_gate_body, nb=nb, c2=C2),
        out_shape=jax.ShapeDtypeStruct((N, C2, HWp), x_nchw.dtype),
        grid=grid,
        in_specs=[
            pl.BlockSpec(memory_space=pltpu.SMEM),
            pl.BlockSpec((nb, C1, HWp), lambda n: (n, 0, 0)),
            pl.BlockSpec((C2 + 8, C1), lambda n: (0, 0)),
            pl.BlockSpec((C2, 1), lambda n: (0, 0)),
        ],
        out_specs=pl.BlockSpec((nb, C2, HWp), lambda n: (n, 0, 0)),
        compiler_params=cparams,
    )(p2k, x3, wa, b1_2d)

    if HWp != HW:
        out3 = out3[:, :, :HW]
    return out3.reshape(N, C2, H, W)


# speedup vs baseline: 3.9586x; 3.5338x over previous
import functools

import jax
import jax.numpy as jnp
from jax.experimental import pallas as pl
from jax.experimental.pallas import tpu as pltpu


def _gate_body(p2_ref, x_ref, w_ref, b_ref, o_ref, *, nb, c1, c2, pt):
    wh = w_ref[...].astype(jnp.bfloat16)
    bias = b_ref[...]
    t_max = p2_ref[0]
    t_sum = p2_ref[1] / jnp.float32(c1)
    t_bias = p2_ref[2]
    hw = x_ref.shape[1]
    dims = (((1,), (1,)), ((), ()))
    for b in range(nb):
        for c in range(hw // pt):
            sl = pl.ds(c * pt, pt)
            x = x_ref[b, sl, :]
            xh = x.astype(jnp.bfloat16)
            y = jax.lax.dot_general(xh, wh, dims,
                                    preferred_element_type=jnp.float32)
            x_max = jnp.max(x, axis=1, keepdims=True)
            x_sum = jnp.sum(x, axis=1, keepdims=True)
            z = (x_max * t_max + x_sum * t_sum + t_bias).astype(jnp.bfloat16)
            zh = jnp.bfloat16(0.5) * z
            z = zh + zh * jnp.tanh(zh)
            gate = (jnp.bfloat16(0.5) + jnp.bfloat16(0.5)
                    * jnp.tanh(jnp.bfloat16(0.5) * z)).astype(jnp.float32)
            h = 0.5 * (y + bias[None, :])
            silu = h + h * jnp.tanh(h)
            o_ref[b, sl, :] = (silu * gate).astype(o_ref.dtype)


def kernel(x_nchw, w1_fold, b1_fold, p2):
    N, C1, H, W = x_nchw.shape
    C2 = w1_fold.shape[0]
    HW = H * W

    xt = x_nchw.reshape(N, C1, HW).transpose(0, 2, 1)
    HWp = ((HW + 7) // 8) * 8
    if HWp != HW:
        xt = jnp.pad(xt, ((0, 0), (0, HWp - HW), (0, 0)))

    nb = 1
    for cand in (8, 4, 2):
        if N % cand == 0 and cand * (C1 + C2) * HWp * 4 <= (24 << 20):
            nb = cand
            break
    pt = 256
    while HWp % pt:
        pt //= 2

    grid = (N // nb,)
    cparams = pltpu.CompilerParams(
        dimension_semantics=("parallel",),
        vmem_limit_bytes=int(56 << 20))

    out_t = pl.pallas_call(
        functools.partial(_gate_body, nb=nb, c1=C1, c2=C2, pt=pt),
        out_shape=jax.ShapeDtypeStruct((N, HWp, C2), x_nchw.dtype),
        grid=grid,
        in_specs=[
            pl.BlockSpec(memory_space=pltpu.SMEM),
            pl.BlockSpec((nb, HWp, C1), lambda n: (n, 0, 0)),
            pl.BlockSpec((C2, C1), lambda n: (0, 0)),
            pl.BlockSpec((C2,), lambda n: (0,)),
        ],
        out_specs=pl.BlockSpec((nb, HWp, C2), lambda n: (n, 0, 0)),
        compiler_params=cparams,
    )(p2, xt, w1_fold, b1_fold)

    if HWp != HW:
        out_t = out_t[:, :HW, :]
    return out_t.transpose(0, 2, 1).reshape(N, C2, H, W)
